# baseline (speedup 1.0000x reference)
import functools

import jax
import jax.numpy as jnp
from jax import lax
from jax.experimental import pallas as pl
from jax.experimental.pallas import tpu as pltpu
from jax.experimental.pallas import tpu_sc as plsc

N = 320000
D = 128
S = 1024
TRASH = S
SACC = S + 8

NC = 2
NS = 16
NW = NC * NS

CH = 128
G = 16
NG = CH // G
NCHUNK = N // CH
NQ = NCHUNK // 4
QBASE = NQ // NW
QEXTRA = NQ - QBASE * NW
SVPAD = 40064
ROWS_PER_TILE = S // NS

T0_GROUPS = 128
T1_GROUPS = 1024

_mesh = plsc.VectorSubcoreMesh(core_axis_name="c", subcore_axis_name="s")


@functools.partial(
    pl.kernel,
    mesh=_mesh,
    out_type=jax.ShapeDtypeStruct((NC, S, D), jnp.float32),
    scratch_types=[
        pltpu.VMEM((CH * D,), jnp.float32),
        pltpu.VMEM((CH * D,), jnp.float32),
        pltpu.VMEM((CH * D,), jnp.float32),
        pltpu.VMEM((CH * D,), jnp.float32),
        pltpu.VMEM((16, D), jnp.float32),
        pltpu.VMEM((16, D), jnp.float32),
        pltpu.VMEM((16, D), jnp.float32),
        pltpu.VMEM((16, D), jnp.float32),
        pltpu.VMEM((80 * 16,), jnp.int32),
        pltpu.VMEM_SHARED((SACC, D), jnp.float32),
        pltpu.SemaphoreType.DMA,
        pltpu.SemaphoreType.DMA,
        pltpu.SemaphoreType.DMA,
        pltpu.SemaphoreType.DMA,
        pltpu.SemaphoreType.DMA,
        pltpu.SemaphoreType.DMA,
        pltpu.SemaphoreType.DMA,
        pltpu.SemaphoreType.DMA,
    ],
)
def _sc_fast(xf_hbm, sv_hbm, zs_hbm, sums_out,
             buf0, buf1, buf2, buf3, red0, red1, red2, red3,
             svbuf, acc, g0, g1, g2, g3, p0, p1, p2, p3):
    c = lax.axis_index("c")
    s = lax.axis_index("s")
    wid = s * NC + c

    bufs = (buf0, buf1, buf2, buf3)
    reds = (red0, red1, red2, red3)
    gsems = (g0, g1, g2, g3)
    psems = (p0, p1, p2, p3)

    nq = QBASE + jnp.where(wid < QEXTRA, 1, 0)
    q0 = wid * QBASE + jnp.minimum(wid, QEXTRA)

    r0 = s * ROWS_PER_TILE
    pltpu.sync_copy(zs_hbm.at[pl.ds(r0, ROWS_PER_TILE)],
                    acc.at[pl.ds(r0, ROWS_PER_TILE)])
    pltpu.sync_copy(sv_hbm.at[pl.ds(q0 * 64, 80 * 16)], svbuf)
    plsc.subcore_barrier()

    def gather_start(cid, e):
        base = pl.multiple_of(cid * (CH * D), 1024)
        pltpu.async_copy(xf_hbm.at[pl.ds(base, CH * D)], bufs[e], gsems[e])

    def gather_wait(cid, e):
        base = pl.multiple_of(cid * (CH * D), 1024)
        pltpu.make_async_copy(xf_hbm.at[pl.ds(base, CH * D)], bufs[e],
                              gsems[e]).wait()

    def psem_drain(e):
        pltpu.make_async_copy(zs_hbm.at[pl.ds(0, 16)], reds[e], psems[e]).wait()

    def quad_body(g, carry):
        for e in range(4):
            cid = (q0 + g) * 4 + e
            l = g * 4 + e

            gather_wait(cid, e)

            @pl.when(g >= 1)
            def _():
                psem_drain(e)

            for u in range(NG):
                def half(it, accs, u=u):
                    base = pl.multiple_of(u * (G * D) + it * (8 * D), 16)
                    out = list(accs)
                    for q in range(8):
                        for d in range(8):
                            out[d] = out[d] + bufs[e][
                                pl.ds(base + q * D + d * 16, 16)]
                    return tuple(out)

                gacc = lax.fori_loop(0, 2, half,
                                     tuple(jnp.zeros((16,), jnp.float32)
                                           for _ in range(8)))
                for d in range(8):
                    reds[e][u, pl.ds(d * 16, 16)] = gacc[d]

            ivec = svbuf[pl.ds(l * 16, 16)]
            pltpu.async_copy(reds[e], acc.at[ivec], psems[e], add=True)

            @pl.when(g + 1 < nq)
            def _():
                gather_start(cid + 4, e)
        return carry

    for e in range(4):
        gather_start(q0 * 4 + e, e)
    lax.fori_loop(0, nq, quad_body, 0)
    for e in range(4):
        psem_drain(e)

    plsc.subcore_barrier()
    pltpu.sync_copy(acc.at[pl.ds(r0, ROWS_PER_TILE)],
                    sums_out.at[c, pl.ds(r0, ROWS_PER_TILE)])


def _make_residue(max_groups):
    rows_per_worker = max_groups * G // NW
    nbatch = rows_per_worker // 16

    @functools.partial(
        pl.kernel,
        mesh=_mesh,
        out_type=jax.ShapeDtypeStruct((NC, S, D), jnp.float32),
        scratch_types=[
            pltpu.VMEM((rows_per_worker,), jnp.int32),
            pltpu.VMEM((rows_per_worker,), jnp.int32),
            pltpu.VMEM((16, D), jnp.float32),
            pltpu.VMEM_SHARED((SACC, D), jnp.float32),
            pltpu.SemaphoreType.DMA,
        ],
    )
    def _sc_residue(x_hbm, rows_hbm, tgt_hbm, zs_hbm, sums_out,
                    rbuf, tbuf, stage, acc, sem):
        c = lax.axis_index("c")
        s = lax.axis_index("s")
        wid = s * NC + c

        r0 = s * ROWS_PER_TILE
        pltpu.sync_copy(zs_hbm.at[pl.ds(r0, ROWS_PER_TILE)],
                        acc.at[pl.ds(r0, ROWS_PER_TILE)])
        base = wid * rows_per_worker
        pltpu.sync_copy(rows_hbm.at[pl.ds(base, rows_per_worker)], rbuf)
        pltpu.sync_copy(tgt_hbm.at[pl.ds(base, rows_per_worker)], tbuf)
        plsc.subcore_barrier()

        def batch(k, carry):
            rvec = rbuf[pl.ds(k * 16, 16)]
            tvec = tbuf[pl.ds(k * 16, 16)]
            pltpu.async_copy(x_hbm.at[rvec], stage, sem).wait()
            pltpu.sync_copy(stage, acc.at[tvec], add=True)
            return carry

        lax.fori_loop(0, nbatch, batch, 0)

        plsc.subcore_barrier()
        pltpu.sync_copy(acc.at[pl.ds(r0, ROWS_PER_TILE)],
                        sums_out.at[c, pl.ds(r0, ROWS_PER_TILE)])

    return _sc_residue


_residue_t0 = _make_residue(T0_GROUPS)
_residue_t1 = _make_residue(T1_GROUPS)


def _tc_head(sf_ref, sr_ref, cnt_ref, g1_ref, be1_ref, w1_ref, b1_ref,
             w2_ref, b2_ref, g2_ref, be2_ref, out_ref):
    sums = (sf_ref[0, :, :] + sf_ref[1, :, :]
            + sr_ref[0, :, :] + sr_ref[1, :, :])
    cnt = jnp.maximum(cnt_ref[:, :], 1.0)
    h = sums / cnt

    def layer_norm(v, gamma, beta):
        mean = jnp.mean(v, axis=-1, keepdims=True)
        var = jnp.var(v, axis=-1, keepdims=True)
        return (v - mean) * lax.rsqrt(var + 1e-5) * gamma + beta

    h = layer_norm(h, g1_ref[0:1, :], be1_ref[0:1, :])
    y = lax.dot_general(h, w1_ref[:, :], (((1,), (1,)), ((), ())),
                        preferred_element_type=jnp.float32,
                        precision=lax.Precision.HIGHEST) + b1_ref[0:1, :]
    y = jnp.where(y > 0, y, jnp.exp(jnp.minimum(y, 0.0)) - 1.0)
    y = lax.dot_general(y, w2_ref[:, :], (((1,), (1,)), ((), ())),
                        preferred_element_type=jnp.float32,
                        precision=lax.Precision.HIGHEST) + b2_ref[0:1, :]
    y = y + h
    out_ref[:, :] = layer_norm(y, g2_ref[0:1, :], be2_ref[0:1, :])


_tc_head_call = pl.pallas_call(
    _tc_head,
    out_shape=jax.ShapeDtypeStruct((S, D), jnp.float32),
)


@jax.jit
def kernel(x, graph_index, gamma1, beta1, W1, b1, W2, b2, gamma2, beta2):
    idx = graph_index.astype(jnp.int32)
    zeros_s = jnp.zeros((S, D), jnp.float32)

    g3 = idx.reshape(NCHUNK, NG, G)
    gf = g3[:, :, 0]
    guni = gf == g3[:, :, G - 1]
    sv = jnp.concatenate(
        [jnp.where(guni, gf, TRASH),
         jnp.full((NCHUNK, 16 - NG), TRASH, jnp.int32)], axis=1).reshape(-1)
    sv = jnp.concatenate(
        [sv, jnp.full((SVPAD - NCHUNK * 16,), TRASH, jnp.int32)])

    rg = jnp.logical_not(guni).reshape(-1)
    nres = jnp.sum(rg)
    gidx = jnp.nonzero(rg, size=T1_GROUPS, fill_value=0)[0]
    valid = jnp.arange(T1_GROUPS) < nres
    rows = (gidx[:, None] * G + jnp.arange(G)[None, :]).astype(jnp.int32)
    tgt = jnp.where(valid[:, None], idx[rows], TRASH).astype(jnp.int32)
    rows = rows.reshape(-1)
    tgt = tgt.reshape(-1)

    ss = jnp.searchsorted(idx, jnp.arange(S + 1, dtype=jnp.int32))
    cnt2d = (ss[1:] - ss[:-1]).astype(jnp.float32).reshape(S, 1)

    sums_f = _sc_fast(x.reshape(N * D), sv, zeros_s)
    sums_r = lax.cond(
        nres <= T0_GROUPS,
        lambda: _residue_t0(x, rows[:T0_GROUPS * G], tgt[:T0_GROUPS * G],
                            zeros_s),
        lambda: _residue_t1(x, rows, tgt, zeros_s))

    return _tc_head_call(
        sums_f, sums_r, cnt2d,
        gamma1.reshape(1, D), beta1.reshape(1, D), W1, b1.reshape(1, D),
        W2, b2.reshape(1, D), gamma2.reshape(1, D), beta2.reshape(1, D))
